# Initial kernel scaffold; baseline (speedup 1.0000x reference)
import jax
import jax.numpy as jnp
from jax.experimental import pallas as pl


def kernel(raw_bbox, roi_scores):
    raise NotImplementedError("write your pallas kernel here")


# speedup vs baseline: 4.2107x; 4.2107x over previous
import functools

import jax
import jax.numpy as jnp
from jax import lax
from jax.experimental import pallas as pl
from jax.experimental.pallas import tpu as pltpu
from jax.experimental.pallas import tpu_sc as plsc

_N_ROI = 5000
_N_CLASS = 21
_NF = _N_CLASS - 1
_TOPK = 1000
_NPAD = 5120
_PK = 1024
_NMS_T = 0.3
_SCORE_T = 0.01
_IMG_W = 800.0
_IMG_H = 800.0

_ITILE = 1024
_NSUB = _ITILE // 128
_NJT = _NPAD // 128


def _rank_body(u_ref, rank_ref):
    it = pl.program_id(1)
    for r in range(_NSUB):
        isub = it * _NSUB + r
        ibase = isub * 128
        ui = u_ref[0, 0, pl.ds(ibase, 128)].reshape(128, 1)
        uim1 = ui - 1

        def jstep(jt, acc):
            uj = u_ref[0, 0, pl.ds(jt * 128, 128)].reshape(1, 128)
            op = jnp.where(jt < isub, uim1, ui)
            return acc + (uj > op).astype(jnp.float32)

        acc = lax.fori_loop(0, _NJT, jstep,
                            jnp.zeros((128, 128), jnp.float32))
        ujd = u_ref[0, 0, pl.ds(ibase, 128)].reshape(1, 128)
        rows = lax.broadcasted_iota(jnp.int32, (128, 128), 0)
        cols = lax.broadcasted_iota(jnp.int32, (128, 128), 1)
        acc = acc + ((ujd == ui) & (cols < rows)).astype(jnp.float32)
        rank_ref[0, r, :] = jnp.sum(acc, axis=1).astype(jnp.int32)


def _ranks(u):
    return pl.pallas_call(
        _rank_body,
        grid=(_NF, _NPAD // _ITILE),
        in_specs=[pl.BlockSpec((1, 1, _NPAD), lambda c, i: (c, 0, 0))],
        out_specs=pl.BlockSpec((1, _NSUB, 128), lambda c, i: (c, i, 0)),
        out_shape=jax.ShapeDtypeStruct((_NF, _NJT, 128), jnp.int32),
    )(u)


@functools.cache
def _make_sc_scatter():
    mesh = plsc.VectorSubcoreMesh(core_axis_name="c", subcore_axis_name="s")

    @functools.partial(
        pl.kernel,
        out_type=(jax.ShapeDtypeStruct((_NF, 8 * _PK), jnp.float32),
                  jax.ShapeDtypeStruct((_NF, 8 * _PK), jnp.float32)),
        mesh=mesh,
        compiler_params=pltpu.CompilerParams(needs_layout_passes=False),
        scratch_types=[
            pltpu.VMEM((_NPAD,), jnp.int32),
            pltpu.VMEM((5, _NPAD), jnp.float32),
            pltpu.VMEM((8 * _PK,), jnp.float32),
            pltpu.VMEM((8 * _PK,), jnp.float32),
        ],
    )
    def sc_scatter(rank_hbm, vals_hbm, out_hbm, out_t_hbm,
                   rnk_v, val_v, out_v, out_t_v):
        wid = lax.axis_index("s") * 2 + lax.axis_index("c")

        @pl.when(wid < _NF)
        def _():
            pltpu.sync_copy(rank_hbm.at[wid], rnk_v)
            pltpu.sync_copy(vals_hbm.at[wid], val_v)

            def step(k, carry):
                idx = rnk_v[pl.ds(k * 16, 16)]
                m = idx < _PK
                safe = jnp.where(m, idx, 0)
                for row in range(5):
                    v = val_v[row, pl.ds(k * 16, 16)]
                    plsc.store_scatter(out_v, [safe + row * _PK], v, mask=m)
                    plsc.store_scatter(out_t_v, [safe * 8 + row], v, mask=m)
                return carry

            lax.fori_loop(0, _NPAD // 16, step, 0)
            pltpu.sync_copy(out_v, out_hbm.at[wid])
            pltpu.sync_copy(out_t_v, out_t_hbm.at[wid])

    return sc_scatter


def _nms_body(srt_ref, srtt_ref, out_ref, m_ref, sup_ref):
    x1 = jnp.clip(srt_ref[0, 0, :], 0.0, _IMG_W)
    y1 = jnp.clip(srt_ref[0, 1, :], 0.0, _IMG_H)
    x2 = jnp.clip(srt_ref[0, 2, :], 0.0, _IMG_W)
    y2 = jnp.clip(srt_ref[0, 3, :], 0.0, _IMG_H)
    s = srt_ref[0, 4, :]
    area = jnp.maximum(x2 - x1, 0.0) * jnp.maximum(y2 - y1, 0.0)

    x1r = x1.reshape(1, _PK)
    y1r = y1.reshape(1, _PK)
    x2r = x2.reshape(1, _PK)
    y2r = y2.reshape(1, _PK)
    ar = area.reshape(1, _PK)

    def build(rt, carry):
        ib = rt * 8
        xi1 = jnp.clip(srtt_ref[0, pl.ds(ib, 8), 0], 0.0, _IMG_W).reshape(8, 1)
        yi1 = jnp.clip(srtt_ref[0, pl.ds(ib, 8), 1], 0.0, _IMG_H).reshape(8, 1)
        xi2 = jnp.clip(srtt_ref[0, pl.ds(ib, 8), 2], 0.0, _IMG_W).reshape(8, 1)
        yi2 = jnp.clip(srtt_ref[0, pl.ds(ib, 8), 3], 0.0, _IMG_H).reshape(8, 1)
        ai = (jnp.maximum(xi2 - xi1, 0.0)
              * jnp.maximum(yi2 - yi1, 0.0))
        ix1 = jnp.maximum(xi1, x1r)
        iy1 = jnp.maximum(yi1, y1r)
        ix2 = jnp.minimum(xi2, x2r)
        iy2 = jnp.minimum(yi2, y2r)
        inter = jnp.maximum(ix2 - ix1, 0.0) * jnp.maximum(iy2 - iy1, 0.0)
        union = ai + ar - inter
        iou = inter / jnp.maximum(union, 1e-9)
        rows = ib + lax.broadcasted_iota(jnp.int32, (8, _PK), 0)
        cols = lax.broadcasted_iota(jnp.int32, (8, _PK), 1)
        m_ref[pl.ds(ib, 8), :] = ((iou > _NMS_T) & (cols < rows)).astype(
            jnp.float32)
        return carry

    lax.fori_loop(0, _PK // 8, build, 0)

    pos = lax.broadcasted_iota(jnp.int32, (_PK,), 0)
    validf = ((s > _SCORE_T) & (pos < _TOPK)).astype(jnp.float32)
    sup_ref[:] = jnp.zeros((_PK,), jnp.float32)

    nblk = _PK // 128
    for k in range(nblk):
        kb0 = k * 128
        supx = sup_ref[pl.ds(kb0, 128)]
        dmat = m_ref[pl.ds(kb0, 128), pl.ds(kb0, 128)]
        vblk = validf.reshape(nblk, 128)[k, :]
        base = vblk * (supx == 0.0).astype(jnp.float32)

        def cond(carry):
            kb, prev = carry
            return jnp.sum(jnp.abs(kb - prev)) > 0.0

        def body(carry):
            kb, _ = carry
            di = jnp.sum(dmat * kb.reshape(1, 128), axis=1)
            return base * (di == 0.0).astype(jnp.float32), kb

        kb, _ = lax.while_loop(
            cond, body, (base, base - 1.0))

        out_ref[0, 5, pl.ds(kb0, 128)] = kb
        kbr = kb.reshape(1, 128)

        def prop(rt, carry):
            mt = m_ref[pl.ds(rt * 128, 128), pl.ds(kb0, 128)]
            contrib = jnp.sum(mt * kbr, axis=1)
            sup_ref[pl.ds(rt * 128, 128)] = (
                sup_ref[pl.ds(rt * 128, 128)] + contrib)
            return carry

        lax.fori_loop(0, nblk, prop, 0)

    keep = out_ref[0, 5, :]
    out_ref[0, 0, :] = x1 * keep
    out_ref[0, 1, :] = y1 * keep
    out_ref[0, 2, :] = x2 * keep
    out_ref[0, 3, :] = y2 * keep
    out_ref[0, 4, :] = s * keep
    zeros = jnp.zeros((_PK,), jnp.float32)
    out_ref[0, 6, :] = zeros
    out_ref[0, 7, :] = zeros


def _nms(sorted_rows, sorted_pos):
    return pl.pallas_call(
        _nms_body,
        grid=(_NF,),
        in_specs=[pl.BlockSpec((1, 8, _PK), lambda c: (c, 0, 0)),
                  pl.BlockSpec((1, _PK, 8), lambda c: (c, 0, 0))],
        out_specs=pl.BlockSpec((1, 8, _PK), lambda c: (c, 0, 0)),
        out_shape=jax.ShapeDtypeStruct((_NF, 8, _PK), jnp.float32),
        scratch_shapes=[
            pltpu.VMEM((_PK, _PK), jnp.float32),
            pltpu.VMEM((_PK,), jnp.float32),
        ],
    )(sorted_rows, sorted_pos)


def kernel(raw_bbox, roi_scores):
    prob = jax.nn.softmax(roi_scores, axis=1)
    s_fg = prob[:, 1:].T
    s_pad = jnp.pad(s_fg, ((0, 0), (0, _NPAD - _N_ROI)))
    keys = lax.bitcast_convert_type(s_pad, jnp.int32)

    boxes = raw_bbox.reshape(_N_ROI, _N_CLASS, 4)[:, 1:, :]
    boxes = boxes.transpose(1, 2, 0)
    boxes = jnp.pad(boxes, ((0, 0), (0, 0), (0, _NPAD - _N_ROI)))
    vals = jnp.concatenate([boxes, s_pad[:, None, :]], axis=1)

    ranks = _ranks(keys.reshape(_NF, 1, _NPAD)).reshape(_NF, _NPAD)
    sorted_flat, sorted_t = _make_sc_scatter()(ranks, vals)
    out = _nms(sorted_flat.reshape(_NF, 8, _PK),
               sorted_t.reshape(_NF, _PK, 8))

    det = out[:, :5, :_TOPK].transpose(0, 2, 1)
    keep = out[:, 5, :_TOPK] > 0.5
    return det, keep


# speedup vs baseline: 13.0276x; 3.0939x over previous
import functools

import jax
import jax.numpy as jnp
from jax import lax
from jax.experimental import pallas as pl
from jax.experimental.pallas import tpu as pltpu
from jax.experimental.pallas import tpu_sc as plsc

_N_ROI = 5000
_N_CLASS = 21
_NF = _N_CLASS - 1
_TOPK = 1000
_NPAD = 5120
_PK = 1024
_NMS_T = 0.3
_SCORE_T = 0.01
_IMG_W = 800.0
_IMG_H = 800.0

_SEL_LEVELS = [(25, 0x7F, 128), (18, 0x7F, 128), (11, 0x7F, 128),
               (4, 0x7F, 128), (0, 0xF, 16)]


def _select_body(u_ref, out_ref):
    nch = u_ref.shape[2] // 128
    p = jnp.int32(0)
    r_want = jnp.int32(_PK)
    cg = jnp.int32(0)

    for lvl, (sh, msk, nb) in enumerate(_SEL_LEVELS):
        bcol = lax.broadcasted_iota(jnp.int32, (nb, 128), 0)
        hi_sh = sh + (7 if msk == 0x7F else 4)

        def chunk(ch, acc, p=p, sh=sh, msk=msk, hi_sh=hi_sh, lvl=lvl):
            uch = u_ref[0, 0, pl.ds(ch * 128, 128)].reshape(1, 128)
            dch = lax.shift_right_logical(uch, sh) & msk
            hit = dch == bcol
            if lvl > 0:
                ok = lax.shift_right_logical(uch, hi_sh) == p
                hit = hit & ok
            return acc + hit.astype(jnp.int32)

        acc = lax.fori_loop(0, nch, chunk, jnp.zeros((nb, 128), jnp.int32))
        hist = jnp.sum(acc, axis=1).reshape(nb, 1)
        bi = lax.broadcasted_iota(jnp.int32, (nb, nb), 0)
        bj = lax.broadcasted_iota(jnp.int32, (nb, nb), 1)
        sfx = jnp.sum(jnp.where(bj > bi, hist.reshape(1, nb), 0),
                      axis=1).reshape(nb, 1)
        pick = (sfx < r_want) & (sfx + hist >= r_want)
        pickf = pick.astype(jnp.int32)
        brow = lax.broadcasted_iota(jnp.int32, (nb, 1), 0)
        bsel = jnp.sum(brow * pickf)
        gsel = jnp.sum(sfx * pickf)
        r_want = r_want - gsel
        cg = cg + gsel
        p = jnp.bitwise_or(lax.shift_left(p, 7 if msk == 0x7F else 4), bsel)

    lane = lax.broadcasted_iota(jnp.int32, (128,), 0)
    out_ref[0, 0, :] = jnp.where(lane == 0, p,
                                 jnp.where(lane == 1, cg, 0))


def _select(u3):
    return pl.pallas_call(
        _select_body,
        grid=(_NF,),
        in_specs=[pl.BlockSpec((1, 1, _NPAD), lambda c: (c, 0, 0))],
        out_specs=pl.BlockSpec((1, 1, 128), lambda c: (c, 0, 0)),
        out_shape=jax.ShapeDtypeStruct((_NF, 1, 128), jnp.int32),
    )(u3)


@functools.cache
def _make_sc_compact():
    mesh = plsc.VectorSubcoreMesh(core_axis_name="c", subcore_axis_name="s")

    @functools.partial(
        pl.kernel,
        out_type=(jax.ShapeDtypeStruct((_NF, _PK), jnp.int32),
                  jax.ShapeDtypeStruct((_NF, 5 * _PK), jnp.float32)),
        mesh=mesh,
        compiler_params=pltpu.CompilerParams(needs_layout_passes=False),
        scratch_types=[
            pltpu.VMEM((_NPAD,), jnp.int32),
            pltpu.VMEM((5, _NPAD), jnp.float32),
            pltpu.VMEM((128,), jnp.int32),
            pltpu.VMEM((_PK,), jnp.int32),
            pltpu.VMEM((5 * _PK,), jnp.float32),
        ],
    )
    def sc_compact(u_hbm, vals_hbm, tb_hbm, cu_hbm, cv_hbm,
                   u_v, val_v, tb_v, cu_v, cv_v):
        wid = lax.axis_index("s") * 2 + lax.axis_index("c")

        @pl.when(wid < _NF)
        def _():
            pltpu.sync_copy(u_hbm.at[wid], u_v)
            pltpu.sync_copy(vals_hbm.at[wid], val_v)
            pltpu.sync_copy(tb_hbm.at[wid], tb_v)
            t_vec = tb_v[pl.ds(0, 16)]
            bud_vec = tb_v[pl.ds(16, 16)]

            def step(ch, carry):
                off_vec, tie = carry
                uch = u_v[pl.ds(ch * 16, 16)]
                m_gt = uch > t_vec
                m_eq = uch == t_vec
                cs_eq = plsc.cumsum(m_eq.astype(jnp.int32))
                sel = m_eq & ((tie + cs_eq) <= bud_vec)
                m = m_gt | sel
                cs_m = plsc.cumsum(m.astype(jnp.int32))
                dest = jnp.minimum(off_vec + cs_m - 1, _PK - 1)
                plsc.store_scatter(cu_v, [dest], uch, mask=m)
                for row in range(5):
                    v = val_v[row, pl.ds(ch * 16, 16)]
                    plsc.store_scatter(cv_v, [dest + row * _PK], v, mask=m)
                off_vec = off_vec + plsc.all_reduce_population_count(m)
                tie = tie + plsc.all_reduce_population_count(sel)
                return off_vec, tie

            lax.fori_loop(0, _NPAD // 16, step,
                          (jnp.zeros((16,), jnp.int32),
                           jnp.zeros((16,), jnp.int32)))
            pltpu.sync_copy(cu_v, cu_hbm.at[wid])
            pltpu.sync_copy(cv_v, cv_hbm.at[wid])

    return sc_compact


def _rank_body(u_ref, rank_ref):
    it = pl.program_id(1)
    njt = u_ref.shape[2] // 128
    for r in range(8):
        isub = it * 8 + r
        ibase = isub * 128
        ui = u_ref[0, 0, pl.ds(ibase, 128)].reshape(128, 1)
        ui_b = jnp.broadcast_to(ui, (128, 128))
        uim1_b = ui_b - 1

        def step_ge(jt, acc):
            uj = u_ref[0, 0, pl.ds(jt * 128, 128)].reshape(1, 128)
            return acc + (uj > uim1_b).astype(jnp.float32)

        def step_gt(jt, acc):
            uj = u_ref[0, 0, pl.ds(jt * 128, 128)].reshape(1, 128)
            return acc + (uj > ui_b).astype(jnp.float32)

        acc = lax.fori_loop(0, isub, step_ge,
                            jnp.zeros((128, 128), jnp.float32))
        acc = lax.fori_loop(isub, njt, step_gt, acc)
        ujd = u_ref[0, 0, pl.ds(ibase, 128)].reshape(1, 128)
        rows = lax.broadcasted_iota(jnp.int32, (128, 128), 0)
        cols = lax.broadcasted_iota(jnp.int32, (128, 128), 1)
        acc = acc + ((ujd == ui) & (cols < rows)).astype(jnp.float32)
        rank_ref[0, r, :] = jnp.sum(acc, axis=1).astype(jnp.int32)


def _ranks(u, npad):
    njt = npad // 128
    return pl.pallas_call(
        _rank_body,
        grid=(_NF, njt // 8),
        in_specs=[pl.BlockSpec((1, 1, npad), lambda c, i: (c, 0, 0))],
        out_specs=pl.BlockSpec((1, 8, 128), lambda c, i: (c, i, 0)),
        out_shape=jax.ShapeDtypeStruct((_NF, njt, 128), jnp.int32),
    )(u)


@functools.cache
def _make_sc_scatter():
    mesh = plsc.VectorSubcoreMesh(core_axis_name="c", subcore_axis_name="s")

    @functools.partial(
        pl.kernel,
        out_type=(jax.ShapeDtypeStruct((_NF, 8 * _PK), jnp.float32),
                  jax.ShapeDtypeStruct((_NF, 8 * _PK), jnp.float32)),
        mesh=mesh,
        compiler_params=pltpu.CompilerParams(needs_layout_passes=False),
        scratch_types=[
            pltpu.VMEM((_PK,), jnp.int32),
            pltpu.VMEM((5 * _PK,), jnp.float32),
            pltpu.VMEM((8 * _PK,), jnp.float32),
            pltpu.VMEM((8 * _PK,), jnp.float32),
        ],
    )
    def sc_scatter(rank_hbm, vals_hbm, out_hbm, out_t_hbm,
                   rnk_v, val_v, out_v, out_t_v):
        wid = lax.axis_index("s") * 2 + lax.axis_index("c")

        @pl.when(wid < _NF)
        def _():
            pltpu.sync_copy(rank_hbm.at[wid], rnk_v)
            pltpu.sync_copy(vals_hbm.at[wid], val_v)

            def step(k, carry):
                idx = rnk_v[pl.ds(k * 16, 16)]
                m = idx < _PK
                safe = jnp.where(m, idx, 0)
                for row in range(5):
                    v = val_v[pl.ds(row * _PK + k * 16, 16)]
                    plsc.store_scatter(out_v, [safe + row * _PK], v, mask=m)
                    plsc.store_scatter(out_t_v, [safe * 8 + row], v, mask=m)
                return carry

            lax.fori_loop(0, _PK // 16, step, 0)
            pltpu.sync_copy(out_v, out_hbm.at[wid])
            pltpu.sync_copy(out_t_v, out_t_hbm.at[wid])

    return sc_scatter


def _nms_body(srt_ref, srtt_ref, out_ref, sup_ref):
    x1 = jnp.clip(srt_ref[0, 0, :], 0.0, _IMG_W)
    y1 = jnp.clip(srt_ref[0, 1, :], 0.0, _IMG_H)
    x2 = jnp.clip(srt_ref[0, 2, :], 0.0, _IMG_W)
    y2 = jnp.clip(srt_ref[0, 3, :], 0.0, _IMG_H)
    s = srt_ref[0, 4, :]
    area = jnp.maximum(x2 - x1, 0.0) * jnp.maximum(y2 - y1, 0.0)

    nblk = _PK // 128

    def col_block(rt):
        rb = rt * 128
        xi1 = jnp.clip(srtt_ref[0, pl.ds(rb, 128), 0],
                       0.0, _IMG_W).reshape(128, 1)
        yi1 = jnp.clip(srtt_ref[0, pl.ds(rb, 128), 1],
                       0.0, _IMG_H).reshape(128, 1)
        xi2 = jnp.clip(srtt_ref[0, pl.ds(rb, 128), 2],
                       0.0, _IMG_W).reshape(128, 1)
        yi2 = jnp.clip(srtt_ref[0, pl.ds(rb, 128), 3],
                       0.0, _IMG_H).reshape(128, 1)
        ai = jnp.maximum(xi2 - xi1, 0.0) * jnp.maximum(yi2 - yi1, 0.0)
        return xi1, yi1, xi2, yi2, ai

    def m_tile(icols, k, diag):
        xi1, yi1, xi2, yi2, ai = icols
        kb0 = k * 128
        x1j = x1[kb0:kb0 + 128].reshape(1, 128)
        y1j = y1[kb0:kb0 + 128].reshape(1, 128)
        x2j = x2[kb0:kb0 + 128].reshape(1, 128)
        y2j = y2[kb0:kb0 + 128].reshape(1, 128)
        aj = area[kb0:kb0 + 128].reshape(1, 128)
        ix1 = jnp.maximum(xi1, x1j)
        iy1 = jnp.maximum(yi1, y1j)
        ix2 = jnp.minimum(xi2, x2j)
        iy2 = jnp.minimum(yi2, y2j)
        inter = jnp.maximum(ix2 - ix1, 0.0) * jnp.maximum(iy2 - iy1, 0.0)
        union = ai + aj - inter
        iou = inter / jnp.maximum(union, 1e-9)
        m = iou > _NMS_T
        if diag:
            rows = lax.broadcasted_iota(jnp.int32, (128, 128), 0)
            cols = lax.broadcasted_iota(jnp.int32, (128, 128), 1)
            m = m & (cols < rows)
        return m.astype(jnp.float32)

    pos = lax.broadcasted_iota(jnp.int32, (_PK,), 0)
    validf = ((s > _SCORE_T) & (pos < _TOPK)).astype(jnp.float32)
    sup_ref[:] = jnp.zeros((_PK,), jnp.float32)

    for k in range(nblk):
        kb0 = k * 128
        supx = sup_ref[pl.ds(kb0, 128)]
        dmat = m_tile(col_block(k), k, diag=True)
        vblk = validf.reshape(nblk, 128)[k, :]
        base = vblk * (supx == 0.0).astype(jnp.float32)

        def cond(carry):
            kb, prev = carry
            return jnp.sum(jnp.abs(kb - prev)) > 0.0

        def body(carry):
            kb, _ = carry
            di = jnp.sum(dmat * kb.reshape(1, 128), axis=1)
            return base * (di == 0.0).astype(jnp.float32), kb

        kb, _ = lax.while_loop(
            cond, body, (base, base - 1.0))

        out_ref[0, 5, pl.ds(kb0, 128)] = kb
        kbr = kb.reshape(1, 128)
        for rt in range(k + 1, nblk):
            mt = m_tile(col_block(rt), k, diag=False)
            contrib = jnp.sum(mt * kbr, axis=1)
            sup_ref[pl.ds(rt * 128, 128)] = (
                sup_ref[pl.ds(rt * 128, 128)] + contrib)

    keep = out_ref[0, 5, :]
    out_ref[0, 0, :] = x1 * keep
    out_ref[0, 1, :] = y1 * keep
    out_ref[0, 2, :] = x2 * keep
    out_ref[0, 3, :] = y2 * keep
    out_ref[0, 4, :] = s * keep
    zeros = jnp.zeros((_PK,), jnp.float32)
    out_ref[0, 6, :] = zeros
    out_ref[0, 7, :] = zeros


def _nms(sorted_rows, sorted_pos):
    return pl.pallas_call(
        _nms_body,
        grid=(_NF,),
        in_specs=[pl.BlockSpec((1, 8, _PK), lambda c: (c, 0, 0)),
                  pl.BlockSpec((1, _PK, 8), lambda c: (c, 0, 0))],
        out_specs=pl.BlockSpec((1, 8, _PK), lambda c: (c, 0, 0)),
        out_shape=jax.ShapeDtypeStruct((_NF, 8, _PK), jnp.float32),
        scratch_shapes=[
            pltpu.VMEM((_PK,), jnp.float32),
        ],
    )(sorted_rows, sorted_pos)


def kernel(raw_bbox, roi_scores):
    prob = jax.nn.softmax(roi_scores, axis=1)
    s_fg = prob[:, 1:].T
    s_pad = jnp.pad(s_fg, ((0, 0), (0, _NPAD - _N_ROI)))
    keys = lax.bitcast_convert_type(s_pad, jnp.int32)

    boxes = raw_bbox.reshape(_N_ROI, _N_CLASS, 4)[:, 1:, :]
    boxes = boxes.transpose(1, 2, 0)
    boxes = jnp.pad(boxes, ((0, 0), (0, 0), (0, _NPAD - _N_ROI)))
    vals = jnp.concatenate([boxes, s_pad[:, None, :]], axis=1)

    sel = _select(keys.reshape(_NF, 1, _NPAD))
    t = sel[:, 0, 0]
    budget = _PK - sel[:, 0, 1]
    tb = jnp.broadcast_to(
        jnp.stack([t, budget], axis=1)[:, :, None], (_NF, 2, 16)
    ).reshape(_NF, 32).astype(jnp.int32)
    tb = jnp.pad(tb, ((0, 0), (0, 96)))

    cand_u, cand_v = _make_sc_compact()(keys, vals, tb)
    rank2 = _ranks(cand_u.reshape(_NF, 1, _PK), _PK).reshape(_NF, _PK)
    sorted_flat, sorted_t = _make_sc_scatter()(rank2, cand_v)
    out = _nms(sorted_flat.reshape(_NF, 8, _PK),
               sorted_t.reshape(_NF, _PK, 8))

    det = out[:, :5, :_TOPK].transpose(0, 2, 1)
    keep = out[:, 5, :_TOPK] > 0.5
    return det, keep


# speedup vs baseline: 17.6099x; 1.3517x over previous
import functools

import jax
import jax.numpy as jnp
from jax import lax
from jax.experimental import pallas as pl
from jax.experimental.pallas import tpu as pltpu
from jax.experimental.pallas import tpu_sc as plsc

_N_ROI = 5000
_N_CLASS = 21
_NF = _N_CLASS - 1
_TOPK = 1000
_NPAD = 5120
_PK = 1024
_NMS_T = 0.3
_SCORE_T = 0.01
_IMG_W = 800.0
_IMG_H = 800.0

_SEL_LEVELS = [(25, 0x7F, 128), (18, 0x7F, 128), (11, 0x7F, 128),
               (4, 0x7F, 128), (0, 0xF, 16)]


def _select_body(u_ref, out_ref):
    nch = u_ref.shape[2] // 128
    p = jnp.int32(0)
    r_want = jnp.int32(_PK)
    cg = jnp.int32(0)

    for lvl, (sh, msk, nb) in enumerate(_SEL_LEVELS):
        bcol = lax.broadcasted_iota(jnp.int32, (nb, 128), 0)
        hi_sh = sh + (7 if msk == 0x7F else 4)

        def chunk(ch, acc, p=p, sh=sh, msk=msk, hi_sh=hi_sh, lvl=lvl):
            uch = u_ref[0, 0, pl.ds(ch * 128, 128)].reshape(1, 128)
            dch = lax.shift_right_logical(uch, sh) & msk
            hit = dch == bcol
            if lvl > 0:
                ok = lax.shift_right_logical(uch, hi_sh) == p
                hit = hit & ok
            return acc + hit.astype(jnp.int32)

        acc = lax.fori_loop(0, nch, chunk, jnp.zeros((nb, 128), jnp.int32))
        hist = jnp.sum(acc, axis=1).reshape(nb, 1)
        bi = lax.broadcasted_iota(jnp.int32, (nb, nb), 0)
        bj = lax.broadcasted_iota(jnp.int32, (nb, nb), 1)
        sfx = jnp.sum(jnp.where(bj > bi, hist.reshape(1, nb), 0),
                      axis=1).reshape(nb, 1)
        pick = (sfx < r_want) & (sfx + hist >= r_want)
        pickf = pick.astype(jnp.int32)
        brow = lax.broadcasted_iota(jnp.int32, (nb, 1), 0)
        bsel = jnp.sum(brow * pickf)
        gsel = jnp.sum(sfx * pickf)
        r_want = r_want - gsel
        cg = cg + gsel
        p = jnp.bitwise_or(lax.shift_left(p, 7 if msk == 0x7F else 4), bsel)

    lane = lax.broadcasted_iota(jnp.int32, (128,), 0)
    out_ref[0, 0, :] = jnp.where(lane == 0, p,
                                 jnp.where(lane == 1, cg, 0))


def _select(u3):
    return pl.pallas_call(
        _select_body,
        grid=(_NF,),
        in_specs=[pl.BlockSpec((1, 1, _NPAD), lambda c: (c, 0, 0))],
        out_specs=pl.BlockSpec((1, 1, 128), lambda c: (c, 0, 0)),
        out_shape=jax.ShapeDtypeStruct((_NF, 1, 128), jnp.int32),
    )(u3)


@functools.cache
def _make_sc_compact():
    mesh = plsc.VectorSubcoreMesh(core_axis_name="c", subcore_axis_name="s")

    @functools.partial(
        pl.kernel,
        out_type=(jax.ShapeDtypeStruct((_NF, _PK), jnp.int32),
                  jax.ShapeDtypeStruct((_NF, 5 * _PK), jnp.float32)),
        mesh=mesh,
        compiler_params=pltpu.CompilerParams(needs_layout_passes=False),
        scratch_types=[
            pltpu.VMEM((_NPAD,), jnp.int32),
            pltpu.VMEM((5, _NPAD), jnp.float32),
            pltpu.VMEM((128,), jnp.int32),
            pltpu.VMEM((_PK,), jnp.int32),
            pltpu.VMEM((5 * _PK,), jnp.float32),
        ],
    )
    def sc_compact(u_hbm, vals_hbm, tb_hbm, cu_hbm, cv_hbm,
                   u_v, val_v, tb_v, cu_v, cv_v):
        wid = lax.axis_index("s") * 2 + lax.axis_index("c")

        @pl.when(wid < _NF)
        def _():
            pltpu.sync_copy(u_hbm.at[wid], u_v)
            pltpu.sync_copy(vals_hbm.at[wid], val_v)
            pltpu.sync_copy(tb_hbm.at[wid], tb_v)
            t_vec = tb_v[pl.ds(0, 16)]
            bud_vec = tb_v[pl.ds(16, 16)]

            def step(ch, carry):
                off_vec, tie = carry
                uch = u_v[pl.ds(ch * 16, 16)]
                m_gt = uch > t_vec
                m_eq = uch == t_vec
                cs_eq = plsc.cumsum(m_eq.astype(jnp.int32))
                sel = m_eq & ((tie + cs_eq) <= bud_vec)
                m = m_gt | sel
                cs_m = plsc.cumsum(m.astype(jnp.int32))
                dest = jnp.minimum(off_vec + cs_m - 1, _PK - 1)
                plsc.store_scatter(cu_v, [dest], uch, mask=m)
                for row in range(5):
                    v = val_v[row, pl.ds(ch * 16, 16)]
                    plsc.store_scatter(cv_v, [dest + row * _PK], v, mask=m)
                off_vec = off_vec + plsc.all_reduce_population_count(m)
                tie = tie + plsc.all_reduce_population_count(sel)
                return off_vec, tie

            lax.fori_loop(0, _NPAD // 16, step,
                          (jnp.zeros((16,), jnp.int32),
                           jnp.zeros((16,), jnp.int32)))
            pltpu.sync_copy(cu_v, cu_hbm.at[wid])
            pltpu.sync_copy(cv_v, cv_hbm.at[wid])

    return sc_compact


def _rank_body(u_ref, rank_ref):
    it = pl.program_id(1)
    njt = u_ref.shape[2] // 128
    for r in range(8):
        isub = it * 8 + r
        ibase = isub * 128
        ui = u_ref[0, 0, pl.ds(ibase, 128)].reshape(128, 1)
        ui_b = jnp.broadcast_to(ui, (128, 128))
        uim1_b = ui_b - 1

        def step_ge(jt, acc):
            uj = u_ref[0, 0, pl.ds(jt * 128, 128)].reshape(1, 128)
            return acc + (uj > uim1_b).astype(jnp.float32)

        def step_gt(jt, acc):
            uj = u_ref[0, 0, pl.ds(jt * 128, 128)].reshape(1, 128)
            return acc + (uj > ui_b).astype(jnp.float32)

        acc = lax.fori_loop(0, isub, step_ge,
                            jnp.zeros((128, 128), jnp.float32))
        acc = lax.fori_loop(isub, njt, step_gt, acc)
        ujd = u_ref[0, 0, pl.ds(ibase, 128)].reshape(1, 128)
        rows = lax.broadcasted_iota(jnp.int32, (128, 128), 0)
        cols = lax.broadcasted_iota(jnp.int32, (128, 128), 1)
        acc = acc + ((ujd == ui) & (cols < rows)).astype(jnp.float32)
        rank_ref[0, r, :] = jnp.sum(acc, axis=1).astype(jnp.int32)


def _ranks(u, npad):
    njt = npad // 128
    return pl.pallas_call(
        _rank_body,
        grid=(_NF, njt // 8),
        in_specs=[pl.BlockSpec((1, 1, npad), lambda c, i: (c, 0, 0))],
        out_specs=pl.BlockSpec((1, 8, 128), lambda c, i: (c, i, 0)),
        out_shape=jax.ShapeDtypeStruct((_NF, njt, 128), jnp.int32),
    )(u)


@functools.cache
def _make_sc_scatter():
    mesh = plsc.VectorSubcoreMesh(core_axis_name="c", subcore_axis_name="s")

    @functools.partial(
        pl.kernel,
        out_type=(jax.ShapeDtypeStruct((_NF, 8 * _PK), jnp.float32),
                  jax.ShapeDtypeStruct((_NF, 8 * _PK), jnp.float32)),
        mesh=mesh,
        compiler_params=pltpu.CompilerParams(needs_layout_passes=False),
        scratch_types=[
            pltpu.VMEM((_PK,), jnp.int32),
            pltpu.VMEM((5 * _PK,), jnp.float32),
            pltpu.VMEM((8 * _PK,), jnp.float32),
            pltpu.VMEM((8 * _PK,), jnp.float32),
        ],
    )
    def sc_scatter(rank_hbm, vals_hbm, out_hbm, out_t_hbm,
                   rnk_v, val_v, out_v, out_t_v):
        wid = lax.axis_index("s") * 2 + lax.axis_index("c")

        @pl.when(wid < _NF)
        def _():
            pltpu.sync_copy(rank_hbm.at[wid], rnk_v)
            pltpu.sync_copy(vals_hbm.at[wid], val_v)

            def step(k, carry):
                idx = rnk_v[pl.ds(k * 16, 16)]
                m = idx < _PK
                safe = jnp.where(m, idx, 0)
                for row in range(5):
                    v = val_v[pl.ds(row * _PK + k * 16, 16)]
                    plsc.store_scatter(out_v, [safe + row * _PK], v, mask=m)
                    plsc.store_scatter(out_t_v, [safe * 8 + row], v, mask=m)
                return carry

            lax.fori_loop(0, _PK // 16, step, 0)
            pltpu.sync_copy(out_v, out_hbm.at[wid])
            pltpu.sync_copy(out_t_v, out_t_hbm.at[wid])

    return sc_scatter


def _nms_body(srt_ref, srtt_ref, out_ref, sup_ref):
    x1 = jnp.clip(srt_ref[0, 0, :], 0.0, _IMG_W)
    y1 = jnp.clip(srt_ref[0, 1, :], 0.0, _IMG_H)
    x2 = jnp.clip(srt_ref[0, 2, :], 0.0, _IMG_W)
    y2 = jnp.clip(srt_ref[0, 3, :], 0.0, _IMG_H)
    s = srt_ref[0, 4, :]
    area = jnp.maximum(x2 - x1, 0.0) * jnp.maximum(y2 - y1, 0.0)

    nblk = _PK // 128

    def col_block(rt):
        rb = rt * 128
        xi1 = jnp.clip(srtt_ref[0, pl.ds(rb, 128), 0],
                       0.0, _IMG_W).reshape(128, 1)
        yi1 = jnp.clip(srtt_ref[0, pl.ds(rb, 128), 1],
                       0.0, _IMG_H).reshape(128, 1)
        xi2 = jnp.clip(srtt_ref[0, pl.ds(rb, 128), 2],
                       0.0, _IMG_W).reshape(128, 1)
        yi2 = jnp.clip(srtt_ref[0, pl.ds(rb, 128), 3],
                       0.0, _IMG_H).reshape(128, 1)
        ai = jnp.maximum(xi2 - xi1, 0.0) * jnp.maximum(yi2 - yi1, 0.0)
        return xi1, yi1, xi2, yi2, ai

    cols_all = [col_block(rt) for rt in range(nblk)]
    rows_all = [
        (x1[k * 128:(k + 1) * 128].reshape(1, 128),
         y1[k * 128:(k + 1) * 128].reshape(1, 128),
         x2[k * 128:(k + 1) * 128].reshape(1, 128),
         y2[k * 128:(k + 1) * 128].reshape(1, 128),
         area[k * 128:(k + 1) * 128].reshape(1, 128))
        for k in range(nblk)
    ]

    def m_tile(icols, k, diag):
        xi1, yi1, xi2, yi2, ai = icols
        x1j, y1j, x2j, y2j, aj = rows_all[k]
        ix1 = jnp.maximum(xi1, x1j)
        iy1 = jnp.maximum(yi1, y1j)
        ix2 = jnp.minimum(xi2, x2j)
        iy2 = jnp.minimum(yi2, y2j)
        inter = jnp.maximum(ix2 - ix1, 0.0) * jnp.maximum(iy2 - iy1, 0.0)
        union = ai + aj - inter
        iou = inter / jnp.maximum(union, 1e-9)
        m = iou > _NMS_T
        if diag:
            rows = lax.broadcasted_iota(jnp.int32, (128, 128), 0)
            cols = lax.broadcasted_iota(jnp.int32, (128, 128), 1)
            m = m & (cols < rows)
        return m.astype(jnp.float32)

    pos = lax.broadcasted_iota(jnp.int32, (_PK,), 0)
    validf = ((s > _SCORE_T) & (pos < _TOPK)).astype(jnp.float32)
    sup_ref[:] = jnp.zeros((_PK,), jnp.float32)

    for k in range(nblk):
        kb0 = k * 128
        supx = sup_ref[pl.ds(kb0, 128)]
        dmat = m_tile(cols_all[k], k, diag=True)
        vblk = validf.reshape(nblk, 128)[k, :]
        base = vblk * (supx == 0.0).astype(jnp.float32)

        def cond(carry):
            kb, prev = carry
            return jnp.sum(jnp.abs(kb - prev)) > 0.0

        def body(carry):
            kb, _ = carry
            di = jnp.sum(dmat * kb.reshape(1, 128), axis=1)
            return base * (di == 0.0).astype(jnp.float32), kb

        kb, _ = lax.while_loop(
            cond, body, (base, base - 1.0))

        out_ref[0, 5, pl.ds(kb0, 128)] = kb
        kbr = kb.reshape(1, 128)
        for rt in range(k + 1, nblk):
            mt = m_tile(cols_all[rt], k, diag=False)
            contrib = jnp.sum(mt * kbr, axis=1)
            sup_ref[pl.ds(rt * 128, 128)] = (
                sup_ref[pl.ds(rt * 128, 128)] + contrib)

    keep = out_ref[0, 5, :]
    out_ref[0, 0, :] = x1 * keep
    out_ref[0, 1, :] = y1 * keep
    out_ref[0, 2, :] = x2 * keep
    out_ref[0, 3, :] = y2 * keep
    out_ref[0, 4, :] = s * keep
    zeros = jnp.zeros((_PK,), jnp.float32)
    out_ref[0, 6, :] = zeros
    out_ref[0, 7, :] = zeros


def _nms(sorted_rows, sorted_pos):
    return pl.pallas_call(
        _nms_body,
        grid=(_NF,),
        in_specs=[pl.BlockSpec((1, 8, _PK), lambda c: (c, 0, 0)),
                  pl.BlockSpec((1, _PK, 8), lambda c: (c, 0, 0))],
        out_specs=pl.BlockSpec((1, 8, _PK), lambda c: (c, 0, 0)),
        out_shape=jax.ShapeDtypeStruct((_NF, 8, _PK), jnp.float32),
        scratch_shapes=[
            pltpu.VMEM((_PK,), jnp.float32),
        ],
    )(sorted_rows, sorted_pos)


def kernel(raw_bbox, roi_scores):
    prob = jax.nn.softmax(roi_scores, axis=1)
    s_fg = prob[:, 1:].T
    s_pad = jnp.pad(s_fg, ((0, 0), (0, _NPAD - _N_ROI)))
    keys = lax.bitcast_convert_type(s_pad, jnp.int32)

    boxes = raw_bbox.reshape(_N_ROI, _N_CLASS, 4)[:, 1:, :]
    boxes = boxes.transpose(1, 2, 0)
    boxes = jnp.pad(boxes, ((0, 0), (0, 0), (0, _NPAD - _N_ROI)))
    vals = jnp.concatenate([boxes, s_pad[:, None, :]], axis=1)

    sel = _select(keys.reshape(_NF, 1, _NPAD))
    t = sel[:, 0, 0]
    budget = _PK - sel[:, 0, 1]
    tb = jnp.broadcast_to(
        jnp.stack([t, budget], axis=1)[:, :, None], (_NF, 2, 16)
    ).reshape(_NF, 32).astype(jnp.int32)
    tb = jnp.pad(tb, ((0, 0), (0, 96)))

    cand_u, cand_v = _make_sc_compact()(keys, vals, tb)
    rank2 = _ranks(cand_u.reshape(_NF, 1, _PK), _PK).reshape(_NF, _PK)
    sorted_flat, sorted_t = _make_sc_scatter()(rank2, cand_v)
    out = _nms(sorted_flat.reshape(_NF, 8, _PK),
               sorted_t.reshape(_NF, _PK, 8))

    det = out[:, :5, :_TOPK].transpose(0, 2, 1)
    keep = out[:, 5, :_TOPK] > 0.5
    return det, keep
